# baseline (speedup 1.0000x reference)
import jax
import jax.numpy as jnp
from jax import lax
from jax.experimental import pallas as pl
from jax.experimental.pallas import tpu as pltpu
from jax.experimental.pallas import tpu_sc as plsc

N_ATOMS = 32768
F_COLS = 9
N_MOLS = 16
VOCAB = 4096
PER_COL_DIM = 32
HIDDEN = 256
SENTINEL = 999999999
NROWS = F_COLS * VOCAB

_INFO = plsc.get_sparse_core_info()
_NC = _INFO.num_cores
_NS = _INFO.num_subcores
NW = _NC * _NS
APW = N_ATOMS // NW
WPM = NW // N_MOLS
CHUNK = 256


def _hist_body(feat_hbm, out_hbm, feat_v, counts_v):
    wid = lax.axis_index("s") * _NC + lax.axis_index("c")
    base = wid * APW

    zf = jnp.zeros((16,), jnp.float32)

    def clr_step(i, _):
        for u in range(8):
            counts_v[pl.ds((i * 8 + u) * 16, 16)] = zf
        return 0

    lax.fori_loop(0, NROWS // (16 * 8), clr_step, 0)

    lanes = lax.iota(jnp.int32, 16)
    ones = jnp.ones((16,), jnp.float32)

    for ch in range(APW // CHUNK):
        pltpu.sync_copy(feat_hbm.at[pl.ds(base + ch * CHUNK, CHUNK), :], feat_v)

        def sc_step(j, _):
            p = j * 16 + lanes
            r = p // F_COLS
            cc = p - r * F_COLS
            v = plsc.load_gather(feat_v, [r, cc])
            v = jnp.where(v >= SENTINEL, 0, v & (VOCAB - 1)) + cc * VOCAB
            plsc.addupdate_scatter(counts_v, [v], ones)
            return 0

        lax.fori_loop(0, CHUNK * F_COLS // 16, sc_step, 0)

    row = (wid % WPM) * N_MOLS + wid // WPM
    pltpu.sync_copy(counts_v, out_hbm.at[row])


def _mm_body(counts_ref, tbl_ref, w_ref, b_ref, lens_ref, out_ref, acc_v):
    k = pl.program_id(0)

    @pl.when(k == 0)
    def _():
        acc_v[...] = jnp.zeros_like(acc_v)

    acc_v[...] += jnp.dot(counts_ref[...], tbl_ref[...],
                          preferred_element_type=jnp.float32)

    @pl.when(k == pl.num_programs(0) - 1)
    def _():
        s = acc_v[pl.ds(0, N_MOLS), :] + acc_v[pl.ds(N_MOLS, N_MOLS), :]
        proj = jnp.dot(s, w_ref[...], preferred_element_type=jnp.float32)
        lens = lens_ref[...]
        denom = jnp.maximum(lens, 1.0)
        out_ref[...] = jnp.where(lens > 0.0, proj / denom + b_ref[...], 0.0)


def kernel(a_features, a_scopes, emb_tables, proj_w, proj_b):
    mesh = plsc.VectorSubcoreMesh(core_axis_name="c", subcore_axis_name="s")
    counts = pl.kernel(
        _hist_body,
        out_type=jax.ShapeDtypeStruct((NW, NROWS), jnp.float32),
        mesh=mesh,
        scratch_types=[
            pltpu.VMEM((CHUNK, F_COLS), jnp.int32),
            pltpu.VMEM((NROWS,), jnp.float32),
        ],
        compiler_params=pltpu.CompilerParams(
            use_tc_tiling_on_sc=False, needs_layout_passes=False),
    )(a_features)

    tbl_flat = emb_tables.reshape(NROWS, PER_COL_DIM)
    lens = a_scopes[:, 1:2].astype(jnp.float32)
    b2d = proj_b.reshape(1, HIDDEN)

    KC = 4096
    grid = (NROWS // KC,)
    return pl.pallas_call(
        _mm_body,
        grid=grid,
        in_specs=[
            pl.BlockSpec((NW, KC), lambda k: (0, k)),
            pl.BlockSpec((KC, PER_COL_DIM), lambda k: (k, 0)),
            pl.BlockSpec((PER_COL_DIM, HIDDEN), lambda k: (0, 0)),
            pl.BlockSpec((1, HIDDEN), lambda k: (0, 0)),
            pl.BlockSpec((N_MOLS, 1), lambda k: (0, 0)),
        ],
        out_specs=pl.BlockSpec((N_MOLS, HIDDEN), lambda k: (0, 0)),
        out_shape=jax.ShapeDtypeStruct((N_MOLS, HIDDEN), jnp.float32),
        scratch_shapes=[pltpu.VMEM((NW, PER_COL_DIM), jnp.float32)],
    )(counts, tbl_flat, proj_w, b2d, lens)


# speedup vs baseline: 1.3944x; 1.3944x over previous
import jax
import jax.numpy as jnp
from jax import lax
from jax.experimental import pallas as pl
from jax.experimental.pallas import tpu as pltpu
from jax.experimental.pallas import tpu_sc as plsc

N_ATOMS = 32768
F_COLS = 9
N_MOLS = 16
VOCAB = 4096
PER_COL_DIM = 32
HIDDEN = 256
SENTINEL = 999999999
NROWS = F_COLS * VOCAB

_INFO = plsc.get_sparse_core_info()
_NC = _INFO.num_cores
_NS = _INFO.num_subcores
NW = _NC * _NS
APW = N_ATOMS // NW
WPM = NW // N_MOLS
CHUNK = 256


def _hist_body(feat_hbm, out_hbm, feat_v, counts_v):
    wid = lax.axis_index("s") * _NC + lax.axis_index("c")
    base = wid * APW

    zf = jnp.zeros((16,), jnp.float32)

    def clr_step(i, _):
        for u in range(8):
            counts_v[pl.ds((i * 8 + u) * 16, 16)] = zf
        return 0

    lax.fori_loop(0, NROWS // (16 * 8), clr_step, 0)

    lanes = lax.iota(jnp.int32, 16)
    ones = jnp.ones((16,), jnp.float32)

    for ch in range(APW // CHUNK):
        pltpu.sync_copy(feat_hbm.at[pl.ds(base + ch * CHUNK, CHUNK), :], feat_v)

        def sc_step(j, _):
            p = j * 16 + lanes
            r = p // F_COLS
            cc = p - r * F_COLS
            v = plsc.load_gather(feat_v, [r, cc])
            v = jnp.where(v >= SENTINEL, 0, v & (VOCAB - 1)) + cc * VOCAB
            plsc.addupdate_scatter(counts_v, [v], ones)
            return 0

        lax.fori_loop(0, CHUNK * F_COLS // 16, sc_step, 0)

    row = (wid % WPM) * N_MOLS + wid // WPM
    pltpu.sync_copy(counts_v, out_hbm.at[row])


def _mm_body(counts_ref, tbl_ref, w_ref, b_ref, lens_ref, out_ref, acc_v):
    k = pl.program_id(0)

    @pl.when(k == 0)
    def _():
        acc_v[...] = jnp.zeros_like(acc_v)

    acc_v[...] += jnp.dot(counts_ref[...], tbl_ref[...],
                          preferred_element_type=jnp.float32)

    @pl.when(k == pl.num_programs(0) - 1)
    def _():
        s = acc_v[pl.ds(0, N_MOLS), :] + acc_v[pl.ds(N_MOLS, N_MOLS), :]
        proj = jnp.dot(s, w_ref[...], preferred_element_type=jnp.float32)
        lens = lens_ref[...]
        denom = jnp.maximum(lens, 1.0)
        out_ref[...] = jnp.where(lens > 0.0, proj / denom + b_ref[...], 0.0)


def kernel(a_features, a_scopes, emb_tables, proj_w, proj_b):
    mesh = plsc.VectorSubcoreMesh(core_axis_name="c", subcore_axis_name="s")
    counts = pl.kernel(
        _hist_body,
        out_type=jax.ShapeDtypeStruct((NW, NROWS), jnp.float32),
        mesh=mesh,
        scratch_types=[
            pltpu.VMEM((CHUNK, F_COLS), jnp.int32),
            pltpu.VMEM((NROWS,), jnp.float32),
        ],
        compiler_params=pltpu.CompilerParams(
            use_tc_tiling_on_sc=True, needs_layout_passes=False),
    )(a_features)

    tbl_flat = emb_tables.reshape(NROWS, PER_COL_DIM)
    lens = a_scopes[:, 1:2].astype(jnp.float32)
    b2d = proj_b.reshape(1, HIDDEN)

    KC = 4096
    grid = (NROWS // KC,)
    return pl.pallas_call(
        _mm_body,
        grid=grid,
        in_specs=[
            pl.BlockSpec((NW, KC), lambda k: (0, k)),
            pl.BlockSpec((KC, PER_COL_DIM), lambda k: (k, 0)),
            pl.BlockSpec((PER_COL_DIM, HIDDEN), lambda k: (0, 0)),
            pl.BlockSpec((1, HIDDEN), lambda k: (0, 0)),
            pl.BlockSpec((N_MOLS, 1), lambda k: (0, 0)),
        ],
        out_specs=pl.BlockSpec((N_MOLS, HIDDEN), lambda k: (0, 0)),
        out_shape=jax.ShapeDtypeStruct((N_MOLS, HIDDEN), jnp.float32),
        scratch_shapes=[pltpu.VMEM((NW, PER_COL_DIM), jnp.float32)],
    )(counts, tbl_flat, proj_w, b2d, lens)


# speedup vs baseline: 1.3979x; 1.0025x over previous
import jax
import jax.numpy as jnp
from jax import lax
from jax.experimental import pallas as pl
from jax.experimental.pallas import tpu as pltpu
from jax.experimental.pallas import tpu_sc as plsc

N_ATOMS = 32768
F_COLS = 9
N_MOLS = 16
VOCAB = 4096
PER_COL_DIM = 32
HIDDEN = 256
SENTINEL = 999999999
NROWS = F_COLS * VOCAB

_INFO = plsc.get_sparse_core_info()
_NC = _INFO.num_cores
_NS = _INFO.num_subcores
NW = _NC * _NS
APW = N_ATOMS // NW
WPM = NW // N_MOLS
CHUNK = 256


def _hist_body(feat_hbm, out_hbm, feat_v, counts_v):
    wid = lax.axis_index("s") * _NC + lax.axis_index("c")
    base = wid * APW

    zf = jnp.zeros((16,), jnp.float32)

    def clr_step(i, _):
        for u in range(8):
            counts_v[pl.ds((i * 8 + u) * 16, 16)] = zf
        return 0

    lax.fori_loop(0, NROWS // (16 * 8), clr_step, 0)

    lanes = lax.iota(jnp.int32, 16)
    ones = jnp.ones((16,), jnp.float32)

    for ch in range(APW // CHUNK):
        pltpu.sync_copy(feat_hbm.at[pl.ds(base + ch * CHUNK, CHUNK), :], feat_v)

        def sc_step(j, _):
            p = j * 16 + lanes
            r = p // F_COLS
            cc = p - r * F_COLS
            v = plsc.load_gather(feat_v, [r, cc])
            v = jnp.where(v >= SENTINEL, 0, v & (VOCAB - 1)) + cc * VOCAB
            plsc.addupdate_scatter(counts_v, [v], ones)
            return 0

        lax.fori_loop(0, CHUNK * F_COLS // 16, sc_step, 0)

    row = (wid % WPM) * N_MOLS + wid // WPM
    pltpu.sync_copy(counts_v, out_hbm.at[row])


def _mm_body(counts_ref, tbl_ref, w_ref, b_ref, lens_ref, out_ref, acc_v):
    k = pl.program_id(0)

    @pl.when(k == 0)
    def _():
        acc_v[...] = jnp.zeros_like(acc_v)

    acc_v[...] += jnp.dot(counts_ref[...], tbl_ref[0],
                          preferred_element_type=jnp.float32)

    @pl.when(k == pl.num_programs(0) - 1)
    def _():
        s = acc_v[pl.ds(0, N_MOLS), :] + acc_v[pl.ds(N_MOLS, N_MOLS), :]
        proj = jnp.dot(s, w_ref[...], preferred_element_type=jnp.float32)
        lens = lens_ref[...]
        denom = jnp.maximum(lens, 1.0)
        out_ref[...] = jnp.where(lens > 0.0, proj / denom + b_ref[...], 0.0)


def kernel(a_features, a_scopes, emb_tables, proj_w, proj_b):
    mesh = plsc.VectorSubcoreMesh(core_axis_name="c", subcore_axis_name="s")
    counts = pl.kernel(
        _hist_body,
        out_type=jax.ShapeDtypeStruct((NW, NROWS), jnp.float32),
        mesh=mesh,
        scratch_types=[
            pltpu.VMEM((CHUNK, F_COLS), jnp.int32),
            pltpu.VMEM((NROWS,), jnp.float32),
        ],
        compiler_params=pltpu.CompilerParams(
            use_tc_tiling_on_sc=True, needs_layout_passes=False),
    )(a_features)

    lens = a_scopes[:, 1:2].astype(jnp.float32)
    b2d = proj_b.reshape(1, HIDDEN)

    KC = VOCAB
    grid = (NROWS // KC,)
    return pl.pallas_call(
        _mm_body,
        grid=grid,
        in_specs=[
            pl.BlockSpec((NW, KC), lambda k: (0, k)),
            pl.BlockSpec((1, KC, PER_COL_DIM), lambda k: (k, 0, 0)),
            pl.BlockSpec((PER_COL_DIM, HIDDEN), lambda k: (0, 0)),
            pl.BlockSpec((1, HIDDEN), lambda k: (0, 0)),
            pl.BlockSpec((N_MOLS, 1), lambda k: (0, 0)),
        ],
        out_specs=pl.BlockSpec((N_MOLS, HIDDEN), lambda k: (0, 0)),
        out_shape=jax.ShapeDtypeStruct((N_MOLS, HIDDEN), jnp.float32),
        scratch_shapes=[pltpu.VMEM((NW, PER_COL_DIM), jnp.float32)],
    )(counts, emb_tables, proj_w, b2d, lens)


# speedup vs baseline: 2.1581x; 1.5438x over previous
import jax
import jax.numpy as jnp
from jax import lax
from jax.experimental import pallas as pl
from jax.experimental.pallas import tpu as pltpu
from jax.experimental.pallas import tpu_sc as plsc

N_ATOMS = 32768
F_COLS = 9
N_MOLS = 16
VOCAB = 4096
PER_COL_DIM = 32
HIDDEN = 256
SENTINEL = 999999999
NROWS = F_COLS * VOCAB

_INFO = plsc.get_sparse_core_info()
_NC = _INFO.num_cores
_NS = _INFO.num_subcores
NW = _NC * _NS
APW = N_ATOMS // NW
WPM = NW // N_MOLS


def _hist_body(feat_hbm, out_hbm, feat_v, counts_v):
    wid = lax.axis_index("s") * _NC + lax.axis_index("c")
    base = wid * APW

    zf = jnp.zeros((16,), jnp.float32)

    def clr_step(i, _):
        for u in range(8):
            counts_v[pl.ds((i * 8 + u) * 16, 16)] = zf
        return 0

    lax.fori_loop(0, NROWS // (16 * 8), clr_step, 0)

    pltpu.sync_copy(feat_hbm.at[:, pl.ds(base, APW)], feat_v)

    ones = jnp.ones((16,), jnp.float32)

    for jj in range(F_COLS):
        def sc_step(j, _, jj=jj):
            v = feat_v[jj, pl.ds(j * 16, 16)]
            v = jnp.where(v >= SENTINEL, 0, v & (VOCAB - 1)) + jj * VOCAB
            plsc.addupdate_scatter(counts_v, [v], ones)
            return 0

        lax.fori_loop(0, APW // 16, sc_step, 0)

    row = (wid % WPM) * N_MOLS + wid // WPM
    pltpu.sync_copy(counts_v, out_hbm.at[row])


def _mm_body(counts_ref, tblt_ref, w_ref, b_ref, lens_ref, out_ref, acc_v):
    k = pl.program_id(0)

    @pl.when(k == 0)
    def _():
        acc_v[...] = jnp.zeros_like(acc_v)

    acc_v[...] += lax.dot_general(
        counts_ref[...], tblt_ref[0],
        (((1,), (1,)), ((), ())),
        preferred_element_type=jnp.float32,
    )

    @pl.when(k == pl.num_programs(0) - 1)
    def _():
        s = acc_v[pl.ds(0, N_MOLS), :] + acc_v[pl.ds(N_MOLS, N_MOLS), :]
        proj = jnp.dot(s, w_ref[...], preferred_element_type=jnp.float32)
        lens = lens_ref[...]
        denom = jnp.maximum(lens, 1.0)
        out_ref[...] = jnp.where(lens > 0.0, proj / denom + b_ref[...], 0.0)


def kernel(a_features, a_scopes, emb_tables, proj_w, proj_b):
    feat_t = a_features.T
    tbl_t = emb_tables.transpose(0, 2, 1)

    mesh = plsc.VectorSubcoreMesh(core_axis_name="c", subcore_axis_name="s")
    counts = pl.kernel(
        _hist_body,
        out_type=jax.ShapeDtypeStruct((NW, NROWS), jnp.float32),
        mesh=mesh,
        scratch_types=[
            pltpu.VMEM((F_COLS, APW), jnp.int32),
            pltpu.VMEM((NROWS,), jnp.float32),
        ],
        compiler_params=pltpu.CompilerParams(
            use_tc_tiling_on_sc=True, needs_layout_passes=False),
    )(feat_t)

    lens = a_scopes[:, 1:2].astype(jnp.float32)
    b2d = proj_b.reshape(1, HIDDEN)

    KC = VOCAB
    grid = (NROWS // KC,)
    return pl.pallas_call(
        _mm_body,
        grid=grid,
        in_specs=[
            pl.BlockSpec((NW, KC), lambda k: (0, k)),
            pl.BlockSpec((1, PER_COL_DIM, KC), lambda k: (k, 0, 0)),
            pl.BlockSpec((PER_COL_DIM, HIDDEN), lambda k: (0, 0)),
            pl.BlockSpec((1, HIDDEN), lambda k: (0, 0)),
            pl.BlockSpec((N_MOLS, 1), lambda k: (0, 0)),
        ],
        out_specs=pl.BlockSpec((N_MOLS, HIDDEN), lambda k: (0, 0)),
        out_shape=jax.ShapeDtypeStruct((N_MOLS, HIDDEN), jnp.float32),
        scratch_shapes=[pltpu.VMEM((NW, PER_COL_DIM), jnp.float32)],
    )(counts, tbl_t, proj_w, b2d, lens)


# speedup vs baseline: 2.1741x; 1.0074x over previous
import jax
import jax.numpy as jnp
from jax import lax
from jax.experimental import pallas as pl
from jax.experimental.pallas import tpu as pltpu
from jax.experimental.pallas import tpu_sc as plsc

N_ATOMS = 32768
F_COLS = 9
N_MOLS = 16
VOCAB = 4096
PER_COL_DIM = 32
HIDDEN = 256
SENTINEL = 999999999
NROWS = F_COLS * VOCAB

_INFO = plsc.get_sparse_core_info()
_NC = _INFO.num_cores
_NS = _INFO.num_subcores
NW = _NC * _NS
APW = N_ATOMS // NW
WPM = NW // N_MOLS


def _hist_body(feat_hbm, out_hbm, feat_v, counts_v):
    wid = lax.axis_index("s") * _NC + lax.axis_index("c")
    base = wid * APW

    zf = jnp.zeros((16,), jnp.float32)

    def clr_step(i, _):
        for u in range(8):
            counts_v[pl.ds((i * 8 + u) * 16, 16)] = zf
        return 0

    lax.fori_loop(0, NROWS // (16 * 8), clr_step, 0)

    pltpu.sync_copy(feat_hbm.at[:, pl.ds(base, APW)], feat_v)

    ones = jnp.ones((16,), jnp.float32)

    def col_step(jj, _):
        def sc_step(j, _):
            v = feat_v[jj, pl.ds(j * 16, 16)]
            v = jnp.where(v >= SENTINEL, 0, v & (VOCAB - 1)) + jj * VOCAB
            plsc.addupdate_scatter(counts_v, [v], ones)
            return 0

        return lax.fori_loop(0, APW // 16, sc_step, 0)

    lax.fori_loop(0, F_COLS, col_step, 0)

    row = (wid % WPM) * N_MOLS + wid // WPM
    pltpu.sync_copy(counts_v, out_hbm.at[row])


def _mm_body(counts_ref, tblt_ref, w_ref, b_ref, lens_ref, out_ref, acc_v):
    k = pl.program_id(0)

    @pl.when(k == 0)
    def _():
        acc_v[...] = jnp.zeros_like(acc_v)

    acc_v[...] += lax.dot_general(
        counts_ref[...], tblt_ref[0],
        (((1,), (1,)), ((), ())),
        preferred_element_type=jnp.float32,
    )

    @pl.when(k == pl.num_programs(0) - 1)
    def _():
        s = acc_v[pl.ds(0, N_MOLS), :] + acc_v[pl.ds(N_MOLS, N_MOLS), :]
        proj = jnp.dot(s, w_ref[...], preferred_element_type=jnp.float32)
        lens = lens_ref[...]
        denom = jnp.maximum(lens, 1.0)
        out_ref[...] = jnp.where(lens > 0.0, proj / denom + b_ref[...], 0.0)


def kernel(a_features, a_scopes, emb_tables, proj_w, proj_b):
    feat_t = a_features.T
    tbl_t = emb_tables.transpose(0, 2, 1)

    mesh = plsc.VectorSubcoreMesh(core_axis_name="c", subcore_axis_name="s")
    counts = pl.kernel(
        _hist_body,
        out_type=jax.ShapeDtypeStruct((NW, NROWS), jnp.float32),
        mesh=mesh,
        scratch_types=[
            pltpu.VMEM((F_COLS, APW), jnp.int32),
            pltpu.VMEM((NROWS,), jnp.float32),
        ],
        compiler_params=pltpu.CompilerParams(
            use_tc_tiling_on_sc=True, needs_layout_passes=False),
    )(feat_t)

    lens = a_scopes[:, 1:2].astype(jnp.float32)
    b2d = proj_b.reshape(1, HIDDEN)

    KC = VOCAB
    grid = (NROWS // KC,)
    return pl.pallas_call(
        _mm_body,
        grid=grid,
        in_specs=[
            pl.BlockSpec((NW, KC), lambda k: (0, k)),
            pl.BlockSpec((1, PER_COL_DIM, KC), lambda k: (k, 0, 0)),
            pl.BlockSpec((PER_COL_DIM, HIDDEN), lambda k: (0, 0)),
            pl.BlockSpec((1, HIDDEN), lambda k: (0, 0)),
            pl.BlockSpec((N_MOLS, 1), lambda k: (0, 0)),
        ],
        out_specs=pl.BlockSpec((N_MOLS, HIDDEN), lambda k: (0, 0)),
        out_shape=jax.ShapeDtypeStruct((N_MOLS, HIDDEN), jnp.float32),
        scratch_shapes=[pltpu.VMEM((NW, PER_COL_DIM), jnp.float32)],
    )(counts, tbl_t, proj_w, b2d, lens)


# speedup vs baseline: 2.2612x; 1.0400x over previous
import jax
import jax.numpy as jnp
from jax import lax
from jax.experimental import pallas as pl
from jax.experimental.pallas import tpu as pltpu
from jax.experimental.pallas import tpu_sc as plsc

N_ATOMS = 32768
F_COLS = 9
N_MOLS = 16
VOCAB = 4096
PER_COL_DIM = 32
HIDDEN = 256
SENTINEL = 999999999
NROWS = F_COLS * VOCAB

_INFO = plsc.get_sparse_core_info()
_NC = _INFO.num_cores
_NS = _INFO.num_subcores
NW = _NC * _NS
APW = N_ATOMS // NW
WPM = NW // N_MOLS


def _hist_body(feat_hbm, out_hbm, feat_v, counts_v, sem):
    wid = lax.axis_index("s") * _NC + lax.axis_index("c")
    base = wid * APW

    cp = pltpu.async_copy(feat_hbm.at[:, pl.ds(base, APW)], feat_v, sem)

    zf = jnp.zeros((16,), jnp.float32)

    def clr_step(i, _):
        for u in range(8):
            counts_v[pl.ds((i * 8 + u) * 16, 16)] = zf
        return 0

    lax.fori_loop(0, NROWS // (16 * 8), clr_step, 0)

    cp.wait()

    ones = jnp.ones((16,), jnp.float32)

    def col_step(jj, _):
        def sc_step(j, _):
            for u in range(2):
                v = feat_v[jj, pl.ds((j * 2 + u) * 16, 16)]
                v = jnp.where(v >= SENTINEL, 0, v & (VOCAB - 1)) + jj * VOCAB
                plsc.addupdate_scatter(counts_v, [v], ones)
            return 0

        return lax.fori_loop(0, APW // 32, sc_step, 0)

    lax.fori_loop(0, F_COLS, col_step, 0)

    row = (wid % WPM) * N_MOLS + wid // WPM
    pltpu.sync_copy(counts_v, out_hbm.at[row])


def _mm_body(counts_ref, tblt_ref, w_ref, b_ref, lens_ref, out_ref, acc_v):
    k = pl.program_id(0)

    @pl.when(k == 0)
    def _():
        acc_v[...] = jnp.zeros_like(acc_v)

    acc_v[...] += lax.dot_general(
        counts_ref[...], tblt_ref[0],
        (((1,), (1,)), ((), ())),
        preferred_element_type=jnp.float32,
    )

    @pl.when(k == pl.num_programs(0) - 1)
    def _():
        s = acc_v[pl.ds(0, N_MOLS), :] + acc_v[pl.ds(N_MOLS, N_MOLS), :]
        proj = jnp.dot(s, w_ref[...], preferred_element_type=jnp.float32)
        lens = lens_ref[...]
        denom = jnp.maximum(lens, 1.0)
        out_ref[...] = jnp.where(lens > 0.0, proj / denom + b_ref[...], 0.0)


def kernel(a_features, a_scopes, emb_tables, proj_w, proj_b):
    feat_t = a_features.T
    tbl_t = emb_tables.transpose(0, 2, 1)

    mesh = plsc.VectorSubcoreMesh(core_axis_name="c", subcore_axis_name="s")
    counts = pl.kernel(
        _hist_body,
        out_type=jax.ShapeDtypeStruct((NW, NROWS), jnp.float32),
        mesh=mesh,
        scratch_types=[
            pltpu.VMEM((F_COLS, APW), jnp.int32),
            pltpu.VMEM((NROWS,), jnp.float32),
            pltpu.SemaphoreType.DMA,
        ],
        compiler_params=pltpu.CompilerParams(
            use_tc_tiling_on_sc=True, needs_layout_passes=False),
    )(feat_t)

    lens = a_scopes[:, 1:2].astype(jnp.float32)
    b2d = proj_b.reshape(1, HIDDEN)

    KC = VOCAB
    grid = (NROWS // KC,)
    return pl.pallas_call(
        _mm_body,
        grid=grid,
        in_specs=[
            pl.BlockSpec((NW, KC), lambda k: (0, k)),
            pl.BlockSpec((1, PER_COL_DIM, KC), lambda k: (k, 0, 0)),
            pl.BlockSpec((PER_COL_DIM, HIDDEN), lambda k: (0, 0)),
            pl.BlockSpec((1, HIDDEN), lambda k: (0, 0)),
            pl.BlockSpec((N_MOLS, 1), lambda k: (0, 0)),
        ],
        out_specs=pl.BlockSpec((N_MOLS, HIDDEN), lambda k: (0, 0)),
        out_shape=jax.ShapeDtypeStruct((N_MOLS, HIDDEN), jnp.float32),
        scratch_shapes=[pltpu.VMEM((NW, PER_COL_DIM), jnp.float32)],
    )(counts, tbl_t, proj_w, b2d, lens)


# speedup vs baseline: 2.5307x; 1.1192x over previous
import jax
import jax.numpy as jnp
from jax import lax
from jax.experimental import pallas as pl
from jax.experimental.pallas import tpu as pltpu
from jax.experimental.pallas import tpu_sc as plsc

N_ATOMS = 32768
F_COLS = 9
N_MOLS = 16
VOCAB = 4096
PER_COL_DIM = 32
HIDDEN = 256
SENTINEL = 999999999
NROWS = F_COLS * VOCAB

_INFO = plsc.get_sparse_core_info()
_NC = _INFO.num_cores
_NS = _INFO.num_subcores
NW = _NC * _NS
APW = N_ATOMS // NW
WPM = NW // N_MOLS


def _hist_body(feat_hbm, out_hbm, feat_v, counts_v, sem):
    wid = lax.axis_index("s") * _NC + lax.axis_index("c")
    base = wid * APW

    cp = pltpu.async_copy(feat_hbm.at[:, pl.ds(base, APW)], feat_v, sem)

    zf = jnp.zeros((16,), jnp.float32)

    def clr_step(i, _):
        for u in range(8):
            counts_v[pl.ds((i * 8 + u) * 16, 16)] = zf
        return 0

    lax.fori_loop(0, NROWS // (16 * 8), clr_step, 0)

    cp.wait()

    ones = jnp.ones((16,), jnp.float32)

    def col_step(jj, _):
        def sc_step(j, _):
            for u in range(2):
                v = feat_v[jj, pl.ds((j * 2 + u) * 16, 16)]
                v = jnp.where(v >= SENTINEL, 0, v & (VOCAB - 1)) + jj * VOCAB
                plsc.addupdate_scatter(counts_v, [v], ones)
            return 0

        return lax.fori_loop(0, APW // 32, sc_step, 0)

    lax.fori_loop(0, F_COLS, col_step, 0)

    row = (wid % WPM) * N_MOLS + wid // WPM
    pltpu.sync_copy(counts_v, out_hbm.at[row])


def _mm_body(counts_ref, tblt_ref, w_ref, b_ref, lens_ref, out_ref, acc_v):
    k = pl.program_id(0)

    @pl.when(k == 0)
    def _():
        acc_v[...] = jnp.zeros_like(acc_v)

    acc_v[...] += lax.dot_general(
        counts_ref[...], tblt_ref[0],
        (((1,), (1,)), ((), ())),
        preferred_element_type=jnp.float32,
    )

    @pl.when(k == pl.num_programs(0) - 1)
    def _():
        s = acc_v[pl.ds(0, N_MOLS), :] + acc_v[pl.ds(N_MOLS, N_MOLS), :]
        proj = jnp.dot(s, w_ref[...], preferred_element_type=jnp.float32)
        lens = lens_ref[...]
        denom = jnp.maximum(lens, 1.0)
        out_ref[...] = jnp.where(lens > 0.0, proj / denom + b_ref[...], 0.0)


def kernel(a_features, a_scopes, emb_tables, proj_w, proj_b):
    feat_t = a_features.T
    tbl_t = emb_tables.transpose(0, 2, 1)

    mesh = plsc.VectorSubcoreMesh(core_axis_name="c", subcore_axis_name="s")
    counts = pl.kernel(
        _hist_body,
        out_type=jax.ShapeDtypeStruct((NW, NROWS), jnp.float32),
        mesh=mesh,
        scratch_types=[
            pltpu.VMEM((F_COLS, APW), jnp.int32),
            pltpu.VMEM((NROWS,), jnp.float32),
            pltpu.SemaphoreType.DMA,
        ],
        compiler_params=pltpu.CompilerParams(
            use_tc_tiling_on_sc=True, needs_layout_passes=False),
    )(feat_t)

    lens = a_scopes[:, 1:2].astype(jnp.float32)
    b2d = proj_b.reshape(1, HIDDEN)

    KC = 3 * VOCAB
    grid = (NROWS // KC,)
    return pl.pallas_call(
        _mm_body,
        grid=grid,
        in_specs=[
            pl.BlockSpec((NW, KC), lambda k: (0, k)),
            pl.BlockSpec((1, PER_COL_DIM, KC), lambda k: (k, 0, 0)),
            pl.BlockSpec((PER_COL_DIM, HIDDEN), lambda k: (0, 0)),
            pl.BlockSpec((1, HIDDEN), lambda k: (0, 0)),
            pl.BlockSpec((N_MOLS, 1), lambda k: (0, 0)),
        ],
        out_specs=pl.BlockSpec((N_MOLS, HIDDEN), lambda k: (0, 0)),
        out_shape=jax.ShapeDtypeStruct((N_MOLS, HIDDEN), jnp.float32),
        scratch_shapes=[pltpu.VMEM((NW, PER_COL_DIM), jnp.float32)],
    )(counts, tbl_t, proj_w, b2d, lens)
